# baseline (speedup 1.0000x reference)
import functools

import jax
import jax.numpy as jnp
from jax import lax
from jax.experimental import pallas as pl
from jax.experimental.pallas import tpu as pltpu
from jax.experimental.pallas import tpu_sc as plsc

B = 4096
L = 200
D = 64
R = B * L
NC, NS, LANES = 2, 16, 16
NW = NC * NS
ROWS_PER_W = R // NW
CHUNK = 128
PERIOD = 3200
CPP = PERIOD // CHUNK
NPER = ROWS_PER_W // PERIOD


def _body(in_hbm, tmpl_hbm, out_hbm, tmpl_v, idx_v, row_v, sem):
    wid = lax.axis_index("s") * NC + lax.axis_index("c")
    wbase = wid * ROWS_PER_W

    pltpu.sync_copy(tmpl_hbm, tmpl_v)

    def period_step(c2, _):
        off = wbase + c2 * PERIOD

        def chunk_step(c1, _):
            for j in range(CHUNK // LANES):
                idx_v[0, pl.ds(j * LANES, LANES)] = (
                    tmpl_v[pl.ds(c1 * CHUNK + j * LANES, LANES)] + off
                )
            pltpu.async_copy(in_hbm.at[idx_v.at[0]], row_v, sem).wait()
            pltpu.sync_copy(
                row_v, out_hbm.at[pl.ds(off + c1 * CHUNK, CHUNK)]
            )
            return _

        lax.fori_loop(0, CPP, chunk_step, None)
        return _

    lax.fori_loop(0, NPER, period_step, None)


@jax.jit
def kernel(inputs, order):
    x = inputs.reshape(R, D)
    tmpl = (
        jnp.arange(PERIOD // L, dtype=jnp.int32)[:, None] * L
        + order[None, :]
    ).reshape(PERIOD)
    mesh = plsc.VectorSubcoreMesh(core_axis_name="c", subcore_axis_name="s")
    k = functools.partial(
        pl.kernel,
        mesh=mesh,
        out_type=jax.ShapeDtypeStruct((R, D), jnp.float32),
        scratch_types=[
            pltpu.VMEM((PERIOD,), jnp.int32),
            pltpu.VMEM((1, CHUNK), jnp.int32),
            pltpu.VMEM((CHUNK, D), jnp.float32),
            pltpu.SemaphoreType.DMA,
        ],
        compiler_params=pltpu.CompilerParams(use_tc_tiling_on_sc=False),
    )(_body)
    out = k(x, tmpl)
    return out.reshape(B, L, D)
